# baseline (speedup 1.0000x reference)
import functools

import jax
import jax.numpy as jnp
from jax import lax
from jax.experimental import pallas as pl
from jax.experimental.pallas import tpu as pltpu
from jax.experimental.pallas import tpu_sc as plsc

_N = 8192
_L = 128
_U = 64

_TC_N = 5120
_TC_R = _TC_N // _L

_SC_N = _N - _TC_N
_NW = 32
_PT = _SC_N // _NW
_NV = _PT // 16
_SC_U = 16


def _vq_tc_kernel(w_ref, x_ref, val_ref, idx_ref):
    xv = x_ref[...]

    def body(t, carry):
        bd, bj, bv = carry
        for u in range(_U):
            j = t * _U + u
            c = w_ref[j]
            d = xv - c
            d = d * d
            m = d < bd
            bd = jnp.where(m, d, bd)
            bj = jnp.where(m, j, bj)
            bv = jnp.where(m, c, bv)
        return bd, bj, bv

    bd0 = jnp.full((_TC_R, _L), jnp.inf, jnp.float32)
    bj0 = jnp.zeros((_TC_R, _L), jnp.int32)
    bv0 = jnp.zeros((_TC_R, _L), jnp.float32)
    _, bj, bv = jax.lax.fori_loop(0, _N // _U, body, (bd0, bj0, bv0))

    idx_ref[...] = bj
    val_ref[...] = bv


def _vq_sc_body(w_hbm, x_hbm, val_hbm, idx_hbm, codes_v, xv_v, valv_v, idxv_v):
    wid = lax.axis_index("s") * 2 + lax.axis_index("c")
    base = wid * _PT
    pltpu.sync_copy(x_hbm.at[pl.ds(base, _PT)], xv_v)
    pltpu.sync_copy(w_hbm, codes_v)
    xs = [xv_v[pl.ds(16 * i, 16)] for i in range(_NV)]

    def body(t, carry):
        bds, bjs, bvs = carry
        cvec = codes_v[pl.ds(t * _SC_U, _SC_U)]
        for u in range(_SC_U):
            j = t * _SC_U + u
            jv = jnp.full((16,), j, jnp.int32)
            c = lax.gather(
                cvec,
                jnp.full((16, 1), u, jnp.int32),
                lax.GatherDimensionNumbers(
                    offset_dims=(),
                    collapsed_slice_dims=(0,),
                    start_index_map=(0,),
                ),
                slice_sizes=(1,),
                mode=lax.GatherScatterMode.PROMISE_IN_BOUNDS,
            )
            bds2, bjs2, bvs2 = [], [], []
            for i in range(_NV):
                d = xs[i] - c
                d = d * d
                m = d < bds[i]
                bds2.append(jnp.where(m, d, bds[i]))
                bjs2.append(jnp.where(m, jv, bjs[i]))
                bvs2.append(jnp.where(m, c, bvs[i]))
            bds, bjs, bvs = bds2, bjs2, bvs2
        return bds, bjs, bvs

    bd0 = [jnp.full((16,), jnp.inf, jnp.float32) for _ in range(_NV)]
    bj0 = [jnp.zeros((16,), jnp.int32) for _ in range(_NV)]
    bv0 = [jnp.zeros((16,), jnp.float32) for _ in range(_NV)]
    _, bj, bv = lax.fori_loop(0, _N // _SC_U, body, (bd0, bj0, bv0))

    for i in range(_NV):
        valv_v[pl.ds(16 * i, 16)] = bv[i]
        idxv_v[pl.ds(16 * i, 16)] = bj[i]
    pltpu.sync_copy(valv_v, val_hbm.at[pl.ds(base, _PT)])
    pltpu.sync_copy(idxv_v, idx_hbm.at[pl.ds(base, _PT)])


_vq_sc = functools.partial(
    pl.kernel,
    out_type=[
        jax.ShapeDtypeStruct((_SC_N,), jnp.float32),
        jax.ShapeDtypeStruct((_SC_N,), jnp.int32),
    ],
    mesh=plsc.VectorSubcoreMesh(core_axis_name="c", subcore_axis_name="s"),
    scratch_types=[
        pltpu.VMEM((_N,), jnp.float32),
        pltpu.VMEM((_PT,), jnp.float32),
        pltpu.VMEM((_PT,), jnp.float32),
        pltpu.VMEM((_PT,), jnp.int32),
    ],
)(_vq_sc_body)


def kernel(x, weight):
    shape = x.shape
    xf = x.reshape(_N)
    wf = weight.reshape(_N)

    val_tc, idx_tc = pl.pallas_call(
        _vq_tc_kernel,
        in_specs=[
            pl.BlockSpec(memory_space=pltpu.MemorySpace.SMEM),
            pl.BlockSpec(memory_space=pltpu.MemorySpace.VMEM),
        ],
        out_specs=[
            pl.BlockSpec(memory_space=pltpu.MemorySpace.VMEM),
            pl.BlockSpec(memory_space=pltpu.MemorySpace.VMEM),
        ],
        out_shape=[
            jax.ShapeDtypeStruct((_TC_R, _L), jnp.float32),
            jax.ShapeDtypeStruct((_TC_R, _L), jnp.int32),
        ],
    )(wf, xf[:_TC_N].reshape(_TC_R, _L))

    val_sc, idx_sc = _vq_sc(wf, xf[_TC_N:])

    val = jnp.concatenate([val_tc.reshape(_TC_N), val_sc])
    idx = jnp.concatenate([idx_tc.reshape(_TC_N), idx_sc])
    return val.reshape(shape), idx.reshape(shape)


# speedup vs baseline: 4.2140x; 4.2140x over previous
import jax
import jax.numpy as jnp
from jax.experimental import pallas as pl
from jax.experimental.pallas import tpu as pltpu

_N = 8192
_R = 64
_L = 128
_U = 128


def _vq_kernel(w_ref, x_ref, val_ref, idx_ref):
    xv = x_ref[...]

    def body(t, carry):
        bd, bj, bv = carry
        for u in range(_U):
            j = t * _U + u
            c = w_ref[j]
            d = xv - c
            d = d * d
            m = d < bd
            bd = jnp.where(m, d, bd)
            bj = jnp.where(m, j, bj)
            bv = jnp.where(m, c, bv)
        return bd, bj, bv

    bd0 = jnp.full((_R, _L), jnp.inf, jnp.float32)
    bj0 = jnp.zeros((_R, _L), jnp.int32)
    bv0 = jnp.zeros((_R, _L), jnp.float32)
    _, bj, bv = jax.lax.fori_loop(0, _N // _U, body, (bd0, bj0, bv0))

    idx_ref[...] = bj
    val_ref[...] = bv


def kernel(x, weight):
    shape = x.shape
    xf = x.reshape(_R, _L)
    wf = weight.reshape(_N)
    val, idx = pl.pallas_call(
        _vq_kernel,
        in_specs=[
            pl.BlockSpec(memory_space=pltpu.MemorySpace.SMEM),
            pl.BlockSpec(memory_space=pltpu.MemorySpace.VMEM),
        ],
        out_specs=[
            pl.BlockSpec(memory_space=pltpu.MemorySpace.VMEM),
            pl.BlockSpec(memory_space=pltpu.MemorySpace.VMEM),
        ],
        out_shape=[
            jax.ShapeDtypeStruct((_R, _L), jnp.float32),
            jax.ShapeDtypeStruct((_R, _L), jnp.int32),
        ],
    )(wf, xf)
    return val.reshape(shape), idx.reshape(shape)
